# baseline (device time: 47174 ns/iter reference)
import jax
import jax.numpy as jnp
from jax import lax
from jax.experimental import pallas as pl
from jax.experimental.pallas import tpu as pltpu

N_DEV = 4


def kernel(x, w_mat):
    m_per, k = x.shape
    _, n_per = w_mat.shape

    def body(x_ref, w_ref, out_ref, comm_ref, send_sems, recv_sems):
        my_pos = lax.axis_index("i")
        left = (my_pos - 1) % N_DEV
        right = (my_pos + 1) % N_DEV

        barrier_sem = pltpu.get_barrier_semaphore()
        for nbr in (left, right):
            pl.semaphore_signal(
                barrier_sem, inc=1,
                device_id=(nbr,), device_id_type=pl.DeviceIdType.MESH,
            )
        pl.semaphore_wait(barrier_sem, 2)

        comm_ref[0] = x_ref[...]
        out_ref[pl.ds(my_pos * m_per, m_per), :] = jnp.dot(
            x_ref[...], w_ref[...], preferred_element_type=jnp.float32
        )

        for h in range(N_DEV - 1):
            rdma = pltpu.make_async_remote_copy(
                src_ref=comm_ref.at[h],
                dst_ref=comm_ref.at[h + 1],
                send_sem=send_sems.at[h],
                recv_sem=recv_sems.at[h],
                device_id=(right,),
                device_id_type=pl.DeviceIdType.MESH,
            )
            rdma.start()
            rdma.wait()

            origin = (my_pos - h - 1) % N_DEV
            out_ref[pl.ds(origin * m_per, m_per), :] = jnp.dot(
                comm_ref[h + 1], w_ref[...], preferred_element_type=jnp.float32
            )

    return pl.pallas_call(
        body,
        out_shape=jax.ShapeDtypeStruct((N_DEV * m_per, n_per), jnp.float32),
        in_specs=[
            pl.BlockSpec(memory_space=pltpu.VMEM),
            pl.BlockSpec(memory_space=pltpu.VMEM),
        ],
        out_specs=pl.BlockSpec(memory_space=pltpu.VMEM),
        scratch_shapes=[
            pltpu.VMEM((N_DEV, m_per, k), x.dtype),
            pltpu.SemaphoreType.DMA((N_DEV - 1,)),
            pltpu.SemaphoreType.DMA((N_DEV - 1,)),
        ],
        compiler_params=pltpu.CompilerParams(collective_id=0),
    )(x, w_mat)


# device time: 27437 ns/iter; 1.7194x vs baseline; 1.7194x over previous
import jax
import jax.numpy as jnp
from jax import lax
from jax.experimental import pallas as pl
from jax.experimental.pallas import tpu as pltpu

N_DEV = 4


def kernel(x, w_mat):
    m_per, k = x.shape
    _, n_per = w_mat.shape
    half = m_per // 2

    def body(x_ref, w_ref, out_ref, buf_l, buf_r, buf_d, sems):
        my_pos = lax.axis_index("i")
        left = (my_pos - 1) % N_DEV
        right = (my_pos + 1) % N_DEV

        barrier_sem = pltpu.get_barrier_semaphore()
        for nbr in (left, right):
            pl.semaphore_signal(
                barrier_sem, inc=1,
                device_id=(nbr,), device_id_type=pl.DeviceIdType.MESH,
            )
        pl.semaphore_wait(barrier_sem, 2)

        send_r = pltpu.make_async_remote_copy(
            src_ref=x_ref, dst_ref=buf_l,
            send_sem=sems.at[0], recv_sem=sems.at[1],
            device_id=(right,), device_id_type=pl.DeviceIdType.MESH,
        )
        send_l = pltpu.make_async_remote_copy(
            src_ref=x_ref, dst_ref=buf_r,
            send_sem=sems.at[2], recv_sem=sems.at[3],
            device_id=(left,), device_id_type=pl.DeviceIdType.MESH,
        )
        send_r.start()
        send_l.start()

        out_ref[pl.ds(my_pos * m_per, m_per), :] = jnp.dot(
            x_ref[...], w_ref[...], preferred_element_type=jnp.float32
        )

        send_r.wait_recv()
        fwd_r = pltpu.make_async_remote_copy(
            src_ref=buf_l.at[pl.ds(0, half), :],
            dst_ref=buf_d.at[pl.ds(0, half), :],
            send_sem=sems.at[4], recv_sem=sems.at[5],
            device_id=(right,), device_id_type=pl.DeviceIdType.MESH,
        )
        fwd_r.start()
        origin_l = (my_pos - 1) % N_DEV
        out_ref[pl.ds(origin_l * m_per, m_per), :] = jnp.dot(
            buf_l[...], w_ref[...], preferred_element_type=jnp.float32
        )

        send_l.wait_recv()
        fwd_l = pltpu.make_async_remote_copy(
            src_ref=buf_r.at[pl.ds(half, half), :],
            dst_ref=buf_d.at[pl.ds(half, half), :],
            send_sem=sems.at[6], recv_sem=sems.at[7],
            device_id=(left,), device_id_type=pl.DeviceIdType.MESH,
        )
        fwd_l.start()
        origin_r = (my_pos + 1) % N_DEV
        out_ref[pl.ds(origin_r * m_per, m_per), :] = jnp.dot(
            buf_r[...], w_ref[...], preferred_element_type=jnp.float32
        )

        fwd_r.wait_recv()
        fwd_l.wait_recv()
        origin_d = (my_pos + 2) % N_DEV
        out_ref[pl.ds(origin_d * m_per, m_per), :] = jnp.dot(
            buf_d[...], w_ref[...], preferred_element_type=jnp.float32
        )

        send_r.wait_send()
        send_l.wait_send()
        fwd_r.wait_send()
        fwd_l.wait_send()

    return pl.pallas_call(
        body,
        out_shape=jax.ShapeDtypeStruct((N_DEV * m_per, n_per), jnp.float32),
        in_specs=[
            pl.BlockSpec(memory_space=pltpu.VMEM),
            pl.BlockSpec(memory_space=pltpu.VMEM),
        ],
        out_specs=pl.BlockSpec(memory_space=pltpu.VMEM),
        scratch_shapes=[
            pltpu.VMEM((m_per, k), x.dtype),
            pltpu.VMEM((m_per, k), x.dtype),
            pltpu.VMEM((m_per, k), x.dtype),
            pltpu.SemaphoreType.DMA((8,)),
        ],
        compiler_params=pltpu.CompilerParams(collective_id=0),
    )(x, w_mat)


# device time: 19924 ns/iter; 2.3677x vs baseline; 1.3771x over previous
import jax
import jax.numpy as jnp
from jax import lax
from jax.experimental import pallas as pl
from jax.experimental.pallas import tpu as pltpu

N_DEV = 4


def kernel(x, w_mat):
    m_per, k = x.shape
    _, n_per = w_mat.shape
    half = m_per // 2

    def body(x_ref, w_ref, out_ref, buf_l, buf_r, buf_d, sems):
        my_pos = lax.axis_index("i")
        left = (my_pos - 1) % N_DEV
        right = (my_pos + 1) % N_DEV

        with jax.named_scope("ph_barrier"):
            barrier_sem = pltpu.get_barrier_semaphore()
            for nbr in (left, right):
                pl.semaphore_signal(
                    barrier_sem, inc=1,
                    device_id=(nbr,), device_id_type=pl.DeviceIdType.MESH,
                )
            pl.semaphore_wait(barrier_sem, 2)

        send_r = pltpu.make_async_remote_copy(
            src_ref=x_ref, dst_ref=buf_l,
            send_sem=sems.at[0], recv_sem=sems.at[1],
            device_id=(right,), device_id_type=pl.DeviceIdType.MESH,
        )
        send_l = pltpu.make_async_remote_copy(
            src_ref=x_ref, dst_ref=buf_r,
            send_sem=sems.at[2], recv_sem=sems.at[3],
            device_id=(left,), device_id_type=pl.DeviceIdType.MESH,
        )
        with jax.named_scope("ph_start_fulls"):
            send_r.start()
            send_l.start()

        with jax.named_scope("ph_gemm_local"):
            out_ref[pl.ds(my_pos * m_per, m_per), :] = jnp.dot(
                x_ref[...], w_ref[...], preferred_element_type=jnp.float32
            )

        with jax.named_scope("ph_wait_left_full"):
            send_r.wait_recv()
        fwd_r = pltpu.make_async_remote_copy(
            src_ref=buf_l.at[pl.ds(0, half), :],
            dst_ref=buf_d.at[pl.ds(0, half), :],
            send_sem=sems.at[4], recv_sem=sems.at[5],
            device_id=(right,), device_id_type=pl.DeviceIdType.MESH,
        )
        with jax.named_scope("ph_fwd_r_start"):
            fwd_r.start()
        with jax.named_scope("ph_gemm_left"):
            origin_l = (my_pos - 1) % N_DEV
            out_ref[pl.ds(origin_l * m_per, m_per), :] = jnp.dot(
                buf_l[...], w_ref[...], preferred_element_type=jnp.float32
            )

        with jax.named_scope("ph_wait_right_full"):
            send_l.wait_recv()
        fwd_l = pltpu.make_async_remote_copy(
            src_ref=buf_r.at[pl.ds(half, half), :],
            dst_ref=buf_d.at[pl.ds(half, half), :],
            send_sem=sems.at[6], recv_sem=sems.at[7],
            device_id=(left,), device_id_type=pl.DeviceIdType.MESH,
        )
        with jax.named_scope("ph_fwd_l_start"):
            fwd_l.start()
        with jax.named_scope("ph_gemm_right"):
            origin_r = (my_pos + 1) % N_DEV
            out_ref[pl.ds(origin_r * m_per, m_per), :] = jnp.dot(
                buf_r[...], w_ref[...], preferred_element_type=jnp.float32
            )

        with jax.named_scope("ph_wait_diag"):
            fwd_r.wait_recv()
            fwd_l.wait_recv()
        with jax.named_scope("ph_gemm_diag"):
            origin_d = (my_pos + 2) % N_DEV
            out_ref[pl.ds(origin_d * m_per, m_per), :] = jnp.dot(
                buf_d[...], w_ref[...], preferred_element_type=jnp.float32
            )

        with jax.named_scope("ph_drain_sends"):
            send_r.wait_send()
            send_l.wait_send()
            fwd_r.wait_send()
            fwd_l.wait_send()

    return pl.pallas_call(
        body,
        out_shape=jax.ShapeDtypeStruct((N_DEV * m_per, n_per), jnp.float32),
        in_specs=[
            pl.BlockSpec(memory_space=pltpu.VMEM),
            pl.BlockSpec(memory_space=pltpu.VMEM),
        ],
        out_specs=pl.BlockSpec(memory_space=pltpu.VMEM),
        scratch_shapes=[
            pltpu.VMEM((m_per, k), x.dtype),
            pltpu.VMEM((m_per, k), x.dtype),
            pltpu.VMEM((m_per, k), x.dtype),
            pltpu.SemaphoreType.DMA((8,)),
        ],
        compiler_params=pltpu.CompilerParams(collective_id=0),
    )(x, w_mat)
